# Initial kernel scaffold; baseline (speedup 1.0000x reference)
import jax
import jax.numpy as jnp
from jax.experimental import pallas as pl


def kernel(hidden_states, router_logits, W1, W3, W2):
    raise NotImplementedError("write your pallas kernel here")


# speedup vs baseline: 1.5349x; 1.5349x over previous
import functools
import jax
import jax.numpy as jnp
from jax.experimental import pallas as pl
from jax.experimental.pallas import tpu as pltpu

E = 8
K = 2
H = 1024
I = 4096
T = 2048
B = 256
NBLK = T * K // B + E
MAXROWS = NBLK * B
CH = 1024
NC = I // CH


def _dispatch(router_logits):
    topv, topi = jax.lax.top_k(router_logits, K)
    w = jax.nn.softmax(topv, axis=-1)
    flat_e = topi.reshape(-1).astype(jnp.int32)
    R = flat_e.shape[0]
    onehot = (flat_e[:, None] == jnp.arange(E, dtype=jnp.int32)[None, :]).astype(jnp.int32)
    ranks = jnp.cumsum(onehot, axis=0)
    myrank = jnp.take_along_axis(ranks, flat_e[:, None], axis=1)[:, 0] - 1
    counts = ranks[-1]
    padded = ((counts + B - 1) // B) * B
    poff = jnp.concatenate([jnp.zeros(1, jnp.int32),
                            jnp.cumsum(padded)[:-1].astype(jnp.int32)])
    dest = poff[flat_e] + myrank
    ends = ((poff + padded) // B).astype(jnp.int32)
    bid = jnp.arange(NBLK, dtype=jnp.int32)
    block_expert = jnp.minimum(
        jnp.sum((bid[:, None] >= ends[None, :]).astype(jnp.int32), axis=1),
        E - 1).astype(jnp.int32)
    nact = (jnp.sum(padded) // B).astype(jnp.int32).reshape(1)
    return dest, w, block_expert, nact


def _mlp_body(be_ref, na_ref, x_ref, w1_ref, w3_ref, w2_ref, out_ref, acc_ref):
    j = pl.program_id(0)
    i = pl.program_id(1)

    @pl.when(i < na_ref[0])
    def _():
        x = x_ref[...]
        g = jax.lax.dot_general(x, w1_ref[0], (((1,), (1,)), ((), ())),
                                preferred_element_type=jnp.float32)
        u = jax.lax.dot_general(x, w3_ref[0], (((1,), (1,)), ((), ())),
                                preferred_element_type=jnp.float32)
        h = g * jax.nn.sigmoid(g) * u
        contrib = jax.lax.dot_general(h, w2_ref[0], (((1,), (1,)), ((), ())),
                                      preferred_element_type=jnp.float32)

        @pl.when(j == 0)
        def _():
            acc_ref[pl.ds(i * B, B), :] = contrib

        @pl.when(j > 0)
        def _():
            acc_ref[pl.ds(i * B, B), :] = acc_ref[pl.ds(i * B, B), :] + contrib

    @pl.when(j == NC - 1)
    def _():
        out_ref[...] = acc_ref[pl.ds(i * B, B), :]


def _grouped_mlp(x_sorted, W1, W3, W2, block_expert, nact):
    grid_spec = pltpu.PrefetchScalarGridSpec(
        num_scalar_prefetch=2,
        grid=(NC, NBLK),
        in_specs=[
            pl.BlockSpec((B, H), lambda j, i, be, na: (i, 0)),
            pl.BlockSpec((1, CH, H), lambda j, i, be, na: (be[i], j, 0)),
            pl.BlockSpec((1, CH, H), lambda j, i, be, na: (be[i], j, 0)),
            pl.BlockSpec((1, H, CH), lambda j, i, be, na: (be[i], 0, j)),
        ],
        out_specs=pl.BlockSpec((B, H), lambda j, i, be, na: (i, 0)),
        scratch_shapes=[pltpu.VMEM((MAXROWS, H), jnp.float32)],
    )
    return pl.pallas_call(
        _mlp_body,
        grid_spec=grid_spec,
        out_shape=jax.ShapeDtypeStruct((MAXROWS, H), jnp.float32),
        compiler_params=pltpu.CompilerParams(
            dimension_semantics=("arbitrary", "arbitrary")),
    )(block_expert, nact, x_sorted, W1, W3, W2)


def kernel(hidden_states, router_logits, W1, W3, W2):
    x = hidden_states.reshape(-1, H)
    dest, w, block_expert, nact = _dispatch(router_logits)
    src_token = jnp.zeros((MAXROWS,), jnp.int32).at[dest].set(
        jnp.arange(T * K, dtype=jnp.int32) // K)
    x_sorted = x[src_token]
    out_rows = _grouped_mlp(x_sorted, W1, W3, W2, block_expert, nact)
    token_rows = dest.reshape(T, K)
    final = (w[:, 0:1] * out_rows[token_rows[:, 0]]
             + w[:, 1:2] * out_rows[token_rows[:, 1]])
    return final.reshape(hidden_states.shape)


# speedup vs baseline: 1.7601x; 1.1467x over previous
import functools
import jax
import jax.numpy as jnp
from jax import lax
from jax.experimental import pallas as pl
from jax.experimental.pallas import tpu as pltpu
from jax.experimental.pallas import tpu_sc as plsc

E = 8
K = 2
H = 1024
I = 4096
T = 2048
B = 256
NBLK = T * K // B + E
MAXROWS = NBLK * B
CH = 1024
NC = I // CH

NCORES = 2
NSUB = 16
NW = NCORES * NSUB
L = 16

_MESH = dict(core_axis_name="c", subcore_axis_name="s",
             num_cores=NCORES, num_subcores=NSUB)
_SC_PARAMS = pltpu.CompilerParams(needs_layout_passes=False)


TPW_A = T // NSUB
NCH_A = TPW_A // L


def _route_body(logits_flat, r0_hbm, r1_hbm, w0_hbm, w1_hbm, meta_hbm,
                lbuf, i1b, i2b, r0b, r1b, w0b, w1b,
                sndbuf, bebuf, allcnt, shared):
    cid = lax.axis_index("c")
    sid = lax.axis_index("s")

    @pl.when(cid == 0)
    def _():
        base_t = sid * TPW_A
        iota = lax.iota(jnp.int32, L)
        zero = jnp.zeros((L,), jnp.int32)
        for e in range(E):
            pltpu.sync_copy(logits_flat.at[pl.ds(e * T + base_t, TPW_A)],
                            lbuf.at[e])

        cnt = zero
        for c in range(NCH_A):
            sl = pl.ds(c * L, L)
            v1 = lbuf[0, sl]
            i1 = zero
            v2 = jnp.full((L,), -jnp.inf, jnp.float32)
            i2 = zero
            for e in range(1, E):
                ve = lbuf[e, sl]
                gt1 = ve > v1
                gt2 = ve > v2
                i2 = jnp.where(gt1, i1, jnp.where(gt2, jnp.int32(e), i2))
                v2 = jnp.where(gt1, v1, jnp.where(gt2, ve, v2))
                i1 = jnp.where(gt1, jnp.int32(e), i1)
                v1 = jnp.where(gt1, ve, v1)
            w0 = 1.0 / (1.0 + jnp.exp(v2 - v1))
            i1b[sl] = i1
            i2b[sl] = i2
            w0b[sl] = w0
            w1b[sl] = 1.0 - w0
            for e in range(E):
                pc = (plsc.all_reduce_population_count(i1 == e)
                      + plsc.all_reduce_population_count(i2 == e))
                cnt = cnt + jnp.where(iota == e, pc, zero)

        sndbuf[...] = cnt
        pltpu.sync_copy(sndbuf, shared.at[sid])
        plsc.subcore_barrier()
        pltpu.sync_copy(shared, allcnt)

        totals = zero
        myprefix = zero
        for w in range(NSUB):
            row = allcnt[w, :]
            totals = totals + row
            myprefix = myprefix + jnp.where(w < sid, row, zero)
        padded = lax.shift_left(
            lax.shift_right_logical(totals + (B - 1), 8), 8)
        poff = plsc.cumsum(padded) - padded
        nact = lax.shift_right_logical(jnp.sum(padded), 8)

        cnt2 = poff + myprefix
        for c in range(NCH_A):
            sl = pl.ds(c * L, L)
            i1 = i1b[sl]
            i2 = i2b[sl]
            r0 = zero
            r1 = zero
            for e in range(E):
                h1 = i1 == e
                h2 = i2 == e
                hi = jnp.where(jnp.logical_or(h1, h2),
                               jnp.int32(1), jnp.int32(0))
                pos = plsc.cumsum(hi) - hi
                d = pos + cnt2[e]
                r0 = jnp.where(h1, d, r0)
                r1 = jnp.where(h2, d, r1)
                cnt2 = cnt2 + jnp.where(iota == e, jnp.sum(hi), zero)
            r0b[sl] = r0
            r1b[sl] = r1

        pltpu.sync_copy(r0b, r0_hbm.at[pl.ds(base_t, TPW_A)])
        pltpu.sync_copy(r1b, r1_hbm.at[pl.ds(base_t, TPW_A)])
        pltpu.sync_copy(w0b, w0_hbm.at[pl.ds(base_t, TPW_A)])
        pltpu.sync_copy(w1b, w1_hbm.at[pl.ds(base_t, TPW_A)])

        @pl.when(sid == 0)
        def _():
            ends = lax.shift_right_logical(poff + padded, 8)
            for bc in range(2):
                lanes = iota + bc * L
                bev = zero
                for e in range(E):
                    bev = bev + jnp.where(lanes >= ends[e],
                                          jnp.int32(1), jnp.int32(0))
                bebuf[pl.ds(bc * L, L)] = jnp.minimum(bev, E - 1)
            bebuf[pl.ds(2 * L, L)] = jnp.where(iota == 0, nact, zero)
            pltpu.sync_copy(bebuf, meta_hbm)


def _route_sc(logits_flat):
    return pl.kernel(
        _route_body,
        out_type=(
            jax.ShapeDtypeStruct((T,), jnp.int32),
            jax.ShapeDtypeStruct((T,), jnp.int32),
            jax.ShapeDtypeStruct((T,), jnp.float32),
            jax.ShapeDtypeStruct((T,), jnp.float32),
            jax.ShapeDtypeStruct((3 * L,), jnp.int32),
        ),
        mesh=plsc.VectorSubcoreMesh(**_MESH),
        scratch_types=[
            pltpu.VMEM((E, TPW_A), jnp.float32),
            pltpu.VMEM((TPW_A,), jnp.int32),
            pltpu.VMEM((TPW_A,), jnp.int32),
            pltpu.VMEM((TPW_A,), jnp.int32),
            pltpu.VMEM((TPW_A,), jnp.int32),
            pltpu.VMEM((TPW_A,), jnp.float32),
            pltpu.VMEM((TPW_A,), jnp.float32),
            pltpu.VMEM((L,), jnp.int32),
            pltpu.VMEM((3 * L,), jnp.int32),
            pltpu.VMEM((NSUB, L), jnp.int32),
            pltpu.VMEM_SHARED((NSUB, L), jnp.int32),
        ],
        compiler_params=_SC_PARAMS,
    )(logits_flat)


TPW_B = T // NW


def _scatter_body(hidden, r0_hbm, r1_hbm, xs_hbm, xbuf, idx, sem):
    cid = lax.axis_index("c")
    sid = lax.axis_index("s")
    base = (sid * NCORES + cid) * TPW_B
    pltpu.sync_copy(hidden.at[pl.ds(base, TPW_B)], xbuf)
    pltpu.sync_copy(r0_hbm.at[pl.ds(base, TPW_B)], idx.at[0])
    pltpu.sync_copy(r1_hbm.at[pl.ds(base, TPW_B)], idx.at[1])
    d0 = pltpu.async_copy(xbuf, xs_hbm.at[idx.at[0]], sem)
    d1 = pltpu.async_copy(xbuf, xs_hbm.at[idx.at[1]], sem)
    d0.wait()
    d1.wait()


def _scatter_sc(x, r0, r1):
    return pl.kernel(
        _scatter_body,
        out_type=jax.ShapeDtypeStruct((MAXROWS, H), jnp.float32),
        mesh=plsc.VectorSubcoreMesh(**_MESH),
        scratch_types=[
            pltpu.VMEM((TPW_B, H), jnp.float32),
            pltpu.VMEM((2, TPW_B), jnp.int32),
            pltpu.SemaphoreType.DMA,
        ],
        compiler_params=_SC_PARAMS,
    )(x, r0, r1)


def _mlp_body(meta_ref, x_ref, w1_ref, w3_ref, w2_ref, out_ref, acc_ref):
    j = pl.program_id(0)
    i = pl.program_id(1)

    @pl.when(i < meta_ref[2 * L])
    def _():
        x = x_ref[...]
        g = lax.dot_general(x, w1_ref[0], (((1,), (1,)), ((), ())),
                            preferred_element_type=jnp.float32)
        u = lax.dot_general(x, w3_ref[0], (((1,), (1,)), ((), ())),
                            preferred_element_type=jnp.float32)
        h = g * jax.nn.sigmoid(g) * u
        contrib = lax.dot_general(h, w2_ref[0], (((1,), (1,)), ((), ())),
                                  preferred_element_type=jnp.float32)

        @pl.when(j == 0)
        def _():
            acc_ref[pl.ds(i * B, B), :] = contrib

        @pl.when(j > 0)
        def _():
            acc_ref[pl.ds(i * B, B), :] = acc_ref[pl.ds(i * B, B), :] + contrib

    @pl.when(j == NC - 1)
    def _():
        out_ref[...] = acc_ref[pl.ds(i * B, B), :]


def _grouped_mlp(x_sorted, W1, W3, W2, meta):
    grid_spec = pltpu.PrefetchScalarGridSpec(
        num_scalar_prefetch=1,
        grid=(NC, NBLK),
        in_specs=[
            pl.BlockSpec((B, H), lambda j, i, m: (i, 0)),
            pl.BlockSpec((1, CH, H), lambda j, i, m: (m[i], j, 0)),
            pl.BlockSpec((1, CH, H), lambda j, i, m: (m[i], j, 0)),
            pl.BlockSpec((1, H, CH), lambda j, i, m: (m[i], 0, j)),
        ],
        out_specs=pl.BlockSpec((B, H), lambda j, i, m: (i, 0)),
        scratch_shapes=[pltpu.VMEM((MAXROWS, H), jnp.float32)],
    )
    return pl.pallas_call(
        _mlp_body,
        grid_spec=grid_spec,
        out_shape=jax.ShapeDtypeStruct((MAXROWS, H), jnp.float32),
        compiler_params=pltpu.CompilerParams(
            dimension_semantics=("arbitrary", "arbitrary")),
    )(meta, x_sorted, W1, W3, W2)


_HB = TPW_B // 2


def _combine_body(rows_hbm, r0_hbm, r1_hbm, w0_hbm, w1_hbm, out_hbm,
                  idx0, idx1, wb0, wb1, buf0, buf1, sem):
    cid = lax.axis_index("c")
    sid = lax.axis_index("s")
    base = (sid * NCORES + cid) * TPW_B
    pltpu.sync_copy(r0_hbm.at[pl.ds(base, TPW_B)], idx0)
    pltpu.sync_copy(r1_hbm.at[pl.ds(base, TPW_B)], idx1)
    pltpu.sync_copy(w0_hbm.at[pl.ds(base, TPW_B)], wb0)
    pltpu.sync_copy(w1_hbm.at[pl.ds(base, TPW_B)], wb1)
    zero = jnp.zeros((L,), jnp.int32)
    for half in range(2):
        d0 = pltpu.async_copy(rows_hbm.at[idx0.at[pl.ds(half * _HB, _HB)]],
                              buf0, sem)
        d1 = pltpu.async_copy(rows_hbm.at[idx1.at[pl.ds(half * _HB, _HB)]],
                              buf1, sem)
        d0.wait()
        d1.wait()

        @pl.loop(0, _HB)
        def _(t):
            idxv = zero + (half * _HB + t)
            w0v = plsc.load_gather(wb0, [idxv])
            w1v = plsc.load_gather(wb1, [idxv])
            for l in range(H // L):
                sl = pl.ds(l * L, L)
                buf0[t, sl] = buf0[t, sl] * w0v + buf1[t, sl] * w1v

        pltpu.sync_copy(buf0, out_hbm.at[pl.ds(base + half * _HB, _HB)])


def _combine_sc(out_rows, r0, r1, w0, w1):
    return pl.kernel(
        _combine_body,
        out_type=jax.ShapeDtypeStruct((T, H), jnp.float32),
        mesh=plsc.VectorSubcoreMesh(**_MESH),
        scratch_types=[
            pltpu.VMEM((TPW_B,), jnp.int32),
            pltpu.VMEM((TPW_B,), jnp.int32),
            pltpu.VMEM((TPW_B,), jnp.float32),
            pltpu.VMEM((TPW_B,), jnp.float32),
            pltpu.VMEM((_HB, H), jnp.float32),
            pltpu.VMEM((_HB, H), jnp.float32),
            pltpu.SemaphoreType.DMA,
        ],
        compiler_params=_SC_PARAMS,
    )(out_rows, r0, r1, w0, w1)


def _dispatch_jnp(router_logits):
    topv, topi = jax.lax.top_k(router_logits, K)
    w = jax.nn.softmax(topv, axis=-1)
    flat_e = topi.reshape(-1).astype(jnp.int32)
    R = flat_e.shape[0]
    onehot = (flat_e[:, None] == jnp.arange(E, dtype=jnp.int32)[None, :]).astype(jnp.int32)
    ranks = jnp.cumsum(onehot, axis=0)
    myrank = jnp.take_along_axis(ranks, flat_e[:, None], axis=1)[:, 0] - 1
    counts = ranks[-1]
    padded = ((counts + B - 1) // B) * B
    poff = jnp.concatenate([jnp.zeros(1, jnp.int32),
                            jnp.cumsum(padded)[:-1].astype(jnp.int32)])
    dest = poff[flat_e] + myrank
    ends = ((poff + padded) // B).astype(jnp.int32)
    bid = jnp.arange(2 * L, dtype=jnp.int32)
    block_expert = jnp.minimum(
        jnp.sum((bid[:, None] >= ends[None, :]).astype(jnp.int32), axis=1),
        E - 1).astype(jnp.int32)
    nact = (jnp.sum(padded) // B).astype(jnp.int32)
    meta = jnp.concatenate([block_expert, jnp.full((L,), nact, jnp.int32)])
    tr = dest.reshape(T, K)
    return tr[:, 0], tr[:, 1], w[:, 0], w[:, 1], meta


def kernel(hidden_states, router_logits, W1, W3, W2):
    x = hidden_states.reshape(-1, H)
    r0, r1, w0, w1, meta = _dispatch_jnp(router_logits)
    x_sorted = _scatter_sc(x, r0, r1)
    out_rows = _grouped_mlp(x_sorted, W1, W3, W2, meta)
    final = _combine_sc(out_rows, r0, r1, w0, w1)
    return final.reshape(hidden_states.shape)


# speedup vs baseline: 1.7894x; 1.0167x over previous
import functools
import jax
import jax.numpy as jnp
from jax import lax
from jax.experimental import pallas as pl
from jax.experimental.pallas import tpu as pltpu
from jax.experimental.pallas import tpu_sc as plsc

E = 8
K = 2
H = 1024
I = 4096
T = 2048
B = 256
NBLK = T * K // B + E
MAXROWS = NBLK * B
CH = 1024
NC = I // CH

NCORES = 2
NSUB = 16
NW = NCORES * NSUB
L = 16

_MESH = dict(core_axis_name="c", subcore_axis_name="s",
             num_cores=NCORES, num_subcores=NSUB)
_SC_PARAMS = pltpu.CompilerParams(needs_layout_passes=False)


TPW_A = T // NW
NCH_A = TPW_A // L
NCHG = T // L


def _top2(lbuf, off):
    zero = jnp.zeros((L,), jnp.int32)
    v1 = lbuf[pl.ds(off, L)]
    i1 = zero
    v2 = jnp.full((L,), -jnp.inf, jnp.float32)
    i2 = zero
    for e in range(1, E):
        ve = lbuf[pl.ds(e * T + off, L)]
        gt1 = ve > v1
        gt2 = ve > v2
        i2 = jnp.where(gt1, i1, jnp.where(gt2, jnp.int32(e), i2))
        v2 = jnp.where(gt1, v1, jnp.where(gt2, ve, v2))
        i1 = jnp.where(gt1, jnp.int32(e), i1)
        v1 = jnp.where(gt1, ve, v1)
    return v1, i1, v2, i2


def _route_body(logits_flat, r0_hbm, r1_hbm, w0_hbm, w1_hbm, meta_hbm,
                lbuf, r0b, r1b, w0b, w1b, bebuf):
    cid = lax.axis_index("c")
    sid = lax.axis_index("s")
    wid = sid * NCORES + cid
    iota = lax.iota(jnp.int32, L)
    zero = jnp.zeros((L,), jnp.int32)
    pltpu.sync_copy(logits_flat, lbuf)

    my_first = wid * NCH_A

    @pl.loop(0, NCHG, init_carry=(zero, zero))
    def scan(c, carry):
        totals, myprefix = carry
        myprefix = jnp.where(c == my_first, totals, myprefix)
        _, i1, _, i2 = _top2(lbuf, c * L)
        for e in range(E):
            s = jnp.sum(jnp.where(jnp.logical_or(i1 == e, i2 == e),
                                  jnp.int32(1), jnp.int32(0)))
            totals = totals + jnp.where(iota == e, s, zero)
        return totals, myprefix

    totals, myprefix = scan
    padded = lax.shift_left(lax.shift_right_logical(totals + (B - 1), 8), 8)
    poff = plsc.cumsum(padded) - padded
    nact = lax.shift_right_logical(jnp.sum(padded), 8)

    cnt2 = poff + myprefix
    for c in range(NCH_A):
        off = (my_first + c) * L
        v1, i1, v2, i2 = _top2(lbuf, off)
        w0 = 1.0 / (1.0 + jnp.exp(v2 - v1))
        sl = pl.ds(c * L, L)
        w0b[sl] = w0
        w1b[sl] = 1.0 - w0
        r0 = zero
        r1 = zero
        for e in range(E):
            h1 = i1 == e
            h2 = i2 == e
            hi = jnp.where(jnp.logical_or(h1, h2),
                           jnp.int32(1), jnp.int32(0))
            pos = plsc.cumsum(hi) - hi
            d = pos + cnt2[e]
            r0 = jnp.where(h1, d, r0)
            r1 = jnp.where(h2, d, r1)
            cnt2 = cnt2 + jnp.where(iota == e, jnp.sum(hi), zero)
        r0b[sl] = r0
        r1b[sl] = r1

    base_t = wid * TPW_A
    pltpu.sync_copy(r0b, r0_hbm.at[pl.ds(base_t, TPW_A)])
    pltpu.sync_copy(r1b, r1_hbm.at[pl.ds(base_t, TPW_A)])
    pltpu.sync_copy(w0b, w0_hbm.at[pl.ds(base_t, TPW_A)])
    pltpu.sync_copy(w1b, w1_hbm.at[pl.ds(base_t, TPW_A)])

    @pl.when(wid == 0)
    def _():
        ends = lax.shift_right_logical(poff + padded, 8)
        for bc in range(2):
            lanes = iota + bc * L
            bev = zero
            for e in range(E):
                bev = bev + jnp.where(lanes >= ends[e],
                                      jnp.int32(1), jnp.int32(0))
            bebuf[pl.ds(bc * L, L)] = jnp.minimum(bev, E - 1)
        bebuf[pl.ds(2 * L, L)] = jnp.where(iota == 0, nact, zero)
        pltpu.sync_copy(bebuf, meta_hbm)


def _route_sc(logits_flat):
    return pl.kernel(
        _route_body,
        out_type=(
            jax.ShapeDtypeStruct((T,), jnp.int32),
            jax.ShapeDtypeStruct((T,), jnp.int32),
            jax.ShapeDtypeStruct((T,), jnp.float32),
            jax.ShapeDtypeStruct((T,), jnp.float32),
            jax.ShapeDtypeStruct((3 * L,), jnp.int32),
        ),
        mesh=plsc.VectorSubcoreMesh(**_MESH),
        scratch_types=[
            pltpu.VMEM((E * T,), jnp.float32),
            pltpu.VMEM((TPW_A,), jnp.int32),
            pltpu.VMEM((TPW_A,), jnp.int32),
            pltpu.VMEM((TPW_A,), jnp.float32),
            pltpu.VMEM((TPW_A,), jnp.float32),
            pltpu.VMEM((3 * L,), jnp.int32),
        ],
        compiler_params=_SC_PARAMS,
    )(logits_flat)


TPW_B = T // NW


def _scatter_body(hidden, r0_hbm, r1_hbm, xs_hbm, xbuf, idx, sem):
    cid = lax.axis_index("c")
    sid = lax.axis_index("s")
    base = (sid * NCORES + cid) * TPW_B
    pltpu.sync_copy(hidden.at[pl.ds(base, TPW_B)], xbuf)
    pltpu.sync_copy(r0_hbm.at[pl.ds(base, TPW_B)], idx.at[0])
    pltpu.sync_copy(r1_hbm.at[pl.ds(base, TPW_B)], idx.at[1])
    d0 = pltpu.async_copy(xbuf, xs_hbm.at[idx.at[0]], sem)
    d1 = pltpu.async_copy(xbuf, xs_hbm.at[idx.at[1]], sem)
    d0.wait()
    d1.wait()


def _scatter_sc(x, r0, r1):
    return pl.kernel(
        _scatter_body,
        out_type=jax.ShapeDtypeStruct((MAXROWS, H), jnp.float32),
        mesh=plsc.VectorSubcoreMesh(**_MESH),
        scratch_types=[
            pltpu.VMEM((TPW_B, H), jnp.float32),
            pltpu.VMEM((2, TPW_B), jnp.int32),
            pltpu.SemaphoreType.DMA,
        ],
        compiler_params=_SC_PARAMS,
    )(x, r0, r1)


def _mlp_body(meta_ref, x_ref, w1_ref, w3_ref, w2_ref, out_ref, acc_ref):
    j = pl.program_id(0)
    i = pl.program_id(1)

    @pl.when(i < meta_ref[2 * L])
    def _():
        x = x_ref[...]
        g = lax.dot_general(x, w1_ref[0], (((1,), (1,)), ((), ())),
                            preferred_element_type=jnp.float32)
        u = lax.dot_general(x, w3_ref[0], (((1,), (1,)), ((), ())),
                            preferred_element_type=jnp.float32)
        h = g * jax.nn.sigmoid(g) * u
        contrib = lax.dot_general(h, w2_ref[0], (((1,), (1,)), ((), ())),
                                  preferred_element_type=jnp.float32)

        @pl.when(j == 0)
        def _():
            acc_ref[pl.ds(i * B, B), :] = contrib

        @pl.when(j > 0)
        def _():
            acc_ref[pl.ds(i * B, B), :] = acc_ref[pl.ds(i * B, B), :] + contrib

    @pl.when(j == NC - 1)
    def _():
        out_ref[...] = acc_ref[pl.ds(i * B, B), :]


def _grouped_mlp(x_sorted, W1, W3, W2, meta):
    grid_spec = pltpu.PrefetchScalarGridSpec(
        num_scalar_prefetch=1,
        grid=(NC, NBLK),
        in_specs=[
            pl.BlockSpec((B, H), lambda j, i, m: (i, 0)),
            pl.BlockSpec((1, CH, H), lambda j, i, m: (m[i], j, 0)),
            pl.BlockSpec((1, CH, H), lambda j, i, m: (m[i], j, 0)),
            pl.BlockSpec((1, H, CH), lambda j, i, m: (m[i], 0, j)),
        ],
        out_specs=pl.BlockSpec((B, H), lambda j, i, m: (i, 0)),
        scratch_shapes=[pltpu.VMEM((MAXROWS, H), jnp.float32)],
    )
    return pl.pallas_call(
        _mlp_body,
        grid_spec=grid_spec,
        out_shape=jax.ShapeDtypeStruct((MAXROWS, H), jnp.float32),
        compiler_params=pltpu.CompilerParams(
            dimension_semantics=("arbitrary", "arbitrary")),
    )(meta, x_sorted, W1, W3, W2)


_HB = TPW_B // 2


def _combine_body(rows_hbm, r0_hbm, r1_hbm, w0_hbm, w1_hbm, out_hbm,
                  idx0, idx1, wb0, wb1, buf0, buf1, sem):
    cid = lax.axis_index("c")
    sid = lax.axis_index("s")
    base = (sid * NCORES + cid) * TPW_B
    pltpu.sync_copy(r0_hbm.at[pl.ds(base, TPW_B)], idx0)
    pltpu.sync_copy(r1_hbm.at[pl.ds(base, TPW_B)], idx1)
    pltpu.sync_copy(w0_hbm.at[pl.ds(base, TPW_B)], wb0)
    pltpu.sync_copy(w1_hbm.at[pl.ds(base, TPW_B)], wb1)
    zero = jnp.zeros((L,), jnp.int32)
    for half in range(2):
        d0 = pltpu.async_copy(rows_hbm.at[idx0.at[pl.ds(half * _HB, _HB)]],
                              buf0, sem)
        d1 = pltpu.async_copy(rows_hbm.at[idx1.at[pl.ds(half * _HB, _HB)]],
                              buf1, sem)
        d0.wait()
        d1.wait()

        @pl.loop(0, _HB)
        def _(t):
            idxv = zero + (half * _HB + t)
            w0v = plsc.load_gather(wb0, [idxv])
            w1v = plsc.load_gather(wb1, [idxv])
            for l in range(H // L):
                sl = pl.ds(l * L, L)
                buf0[t, sl] = buf0[t, sl] * w0v + buf1[t, sl] * w1v

        pltpu.sync_copy(buf0, out_hbm.at[pl.ds(base + half * _HB, _HB)])


def _combine_sc(out_rows, r0, r1, w0, w1):
    return pl.kernel(
        _combine_body,
        out_type=jax.ShapeDtypeStruct((T, H), jnp.float32),
        mesh=plsc.VectorSubcoreMesh(**_MESH),
        scratch_types=[
            pltpu.VMEM((TPW_B,), jnp.int32),
            pltpu.VMEM((TPW_B,), jnp.int32),
            pltpu.VMEM((TPW_B,), jnp.float32),
            pltpu.VMEM((TPW_B,), jnp.float32),
            pltpu.VMEM((_HB, H), jnp.float32),
            pltpu.VMEM((_HB, H), jnp.float32),
            pltpu.SemaphoreType.DMA,
        ],
        compiler_params=_SC_PARAMS,
    )(out_rows, r0, r1, w0, w1)


def _dispatch_jnp(router_logits):
    topv, topi = jax.lax.top_k(router_logits, K)
    w = jax.nn.softmax(topv, axis=-1)
    flat_e = topi.reshape(-1).astype(jnp.int32)
    R = flat_e.shape[0]
    onehot = (flat_e[:, None] == jnp.arange(E, dtype=jnp.int32)[None, :]).astype(jnp.int32)
    ranks = jnp.cumsum(onehot, axis=0)
    myrank = jnp.take_along_axis(ranks, flat_e[:, None], axis=1)[:, 0] - 1
    counts = ranks[-1]
    padded = ((counts + B - 1) // B) * B
    poff = jnp.concatenate([jnp.zeros(1, jnp.int32),
                            jnp.cumsum(padded)[:-1].astype(jnp.int32)])
    dest = poff[flat_e] + myrank
    ends = ((poff + padded) // B).astype(jnp.int32)
    bid = jnp.arange(2 * L, dtype=jnp.int32)
    block_expert = jnp.minimum(
        jnp.sum((bid[:, None] >= ends[None, :]).astype(jnp.int32), axis=1),
        E - 1).astype(jnp.int32)
    nact = (jnp.sum(padded) // B).astype(jnp.int32)
    meta = jnp.concatenate([block_expert, jnp.full((L,), nact, jnp.int32)])
    tr = dest.reshape(T, K)
    return tr[:, 0], tr[:, 1], w[:, 0], w[:, 1], meta


def kernel(hidden_states, router_logits, W1, W3, W2):
    x = hidden_states.reshape(-1, H)
    logits_flat = router_logits.T.reshape(-1)
    r0, r1, w0, w1, meta = _route_sc(logits_flat)
    x_sorted = _scatter_sc(x, r0, r1)
    out_rows = _grouped_mlp(x_sorted, W1, W3, W2, meta)
    final = _combine_sc(out_rows, r0, r1, w0, w1)
    return final.reshape(hidden_states.shape)


# speedup vs baseline: 1.9126x; 1.0689x over previous
import functools
import jax
import jax.numpy as jnp
from jax import lax
from jax.experimental import pallas as pl
from jax.experimental.pallas import tpu as pltpu
from jax.experimental.pallas import tpu_sc as plsc

E = 8
K = 2
H = 1024
I = 4096
T = 2048
B = 256
NBLK = T * K // B + E
MAXROWS = NBLK * B
CH = 1024
NC = I // CH

NCORES = 2
NSUB = 16
NW = NCORES * NSUB
L = 16

_MESH = dict(core_axis_name="c", subcore_axis_name="s",
             num_cores=NCORES, num_subcores=NSUB)
_SC_PARAMS = pltpu.CompilerParams(needs_layout_passes=False)


TPW_A = T // NW
NCH_A = TPW_A // L
NCHG = T // L


def _top2(lbuf, off):
    zero = jnp.zeros((L,), jnp.int32)
    v1 = lbuf[pl.ds(off, L)]
    i1 = zero
    v2 = jnp.full((L,), -jnp.inf, jnp.float32)
    i2 = zero
    for e in range(1, E):
        ve = lbuf[pl.ds(e * T + off, L)]
        gt1 = ve > v1
        gt2 = ve > v2
        i2 = jnp.where(gt1, i1, jnp.where(gt2, jnp.int32(e), i2))
        v2 = jnp.where(gt1, v1, jnp.where(gt2, ve, v2))
        i1 = jnp.where(gt1, jnp.int32(e), i1)
        v1 = jnp.where(gt1, ve, v1)
    return v1, i1, v2, i2


def _route_body(logits_flat, r0_hbm, r1_hbm, w0_hbm, w1_hbm, meta_hbm,
                lbuf, r0b, r1b, w0b, w1b, bebuf):
    cid = lax.axis_index("c")
    sid = lax.axis_index("s")
    wid = sid * NCORES + cid
    iota = lax.iota(jnp.int32, L)
    zero = jnp.zeros((L,), jnp.int32)
    pltpu.sync_copy(logits_flat, lbuf)

    my_first = wid * NCH_A

    @pl.loop(0, NCHG, init_carry=(zero, zero))
    def scan(c, carry):
        totals, myprefix = carry
        myprefix = jnp.where(c == my_first, totals, myprefix)
        _, i1, _, i2 = _top2(lbuf, c * L)
        for e in range(E):
            s = jnp.sum(jnp.where(jnp.logical_or(i1 == e, i2 == e),
                                  jnp.int32(1), jnp.int32(0)))
            totals = totals + jnp.where(iota == e, s, zero)
        return totals, myprefix

    totals, myprefix = scan
    padded = lax.shift_left(lax.shift_right_logical(totals + (B - 1), 8), 8)
    poff = plsc.cumsum(padded) - padded
    nact = lax.shift_right_logical(jnp.sum(padded), 8)

    cnt2 = poff + myprefix
    for c in range(NCH_A):
        off = (my_first + c) * L
        v1, i1, v2, i2 = _top2(lbuf, off)
        w0 = 1.0 / (1.0 + jnp.exp(v2 - v1))
        sl = pl.ds(c * L, L)
        w0b[sl] = w0
        w1b[sl] = 1.0 - w0
        r0 = zero
        r1 = zero
        for e in range(E):
            h1 = i1 == e
            h2 = i2 == e
            hi = jnp.where(jnp.logical_or(h1, h2),
                           jnp.int32(1), jnp.int32(0))
            pos = plsc.cumsum(hi) - hi
            d = pos + cnt2[e]
            r0 = jnp.where(h1, d, r0)
            r1 = jnp.where(h2, d, r1)
            cnt2 = cnt2 + jnp.where(iota == e, jnp.sum(hi), zero)
        r0b[sl] = r0
        r1b[sl] = r1

    base_t = wid * TPW_A
    pltpu.sync_copy(r0b, r0_hbm.at[pl.ds(base_t, TPW_A)])
    pltpu.sync_copy(r1b, r1_hbm.at[pl.ds(base_t, TPW_A)])
    pltpu.sync_copy(w0b, w0_hbm.at[pl.ds(base_t, TPW_A)])
    pltpu.sync_copy(w1b, w1_hbm.at[pl.ds(base_t, TPW_A)])

    @pl.when(wid == 0)
    def _():
        ends = lax.shift_right_logical(poff + padded, 8)
        for bc in range(2):
            lanes = iota + bc * L
            bev = zero
            for e in range(E):
                bev = bev + jnp.where(lanes >= ends[e],
                                      jnp.int32(1), jnp.int32(0))
            bebuf[pl.ds(bc * L, L)] = jnp.minimum(bev, E - 1)
        bebuf[pl.ds(2 * L, L)] = jnp.where(iota == 0, nact, zero)
        pltpu.sync_copy(bebuf, meta_hbm)


def _route_sc(logits_flat):
    return pl.kernel(
        _route_body,
        out_type=(
            jax.ShapeDtypeStruct((T,), jnp.int32),
            jax.ShapeDtypeStruct((T,), jnp.int32),
            jax.ShapeDtypeStruct((T,), jnp.float32),
            jax.ShapeDtypeStruct((T,), jnp.float32),
            jax.ShapeDtypeStruct((3 * L,), jnp.int32),
        ),
        mesh=plsc.VectorSubcoreMesh(**_MESH),
        scratch_types=[
            pltpu.VMEM((E * T,), jnp.float32),
            pltpu.VMEM((TPW_A,), jnp.int32),
            pltpu.VMEM((TPW_A,), jnp.int32),
            pltpu.VMEM((TPW_A,), jnp.float32),
            pltpu.VMEM((TPW_A,), jnp.float32),
            pltpu.VMEM((3 * L,), jnp.int32),
        ],
        compiler_params=_SC_PARAMS,
    )(logits_flat)


TPW_B = T // NW


def _scatter_body(hidden, r0_hbm, r1_hbm, xs_hbm, xbuf, idx, sem):
    cid = lax.axis_index("c")
    sid = lax.axis_index("s")
    base = (sid * NCORES + cid) * TPW_B
    pltpu.sync_copy(hidden.at[pl.ds(base, TPW_B)], xbuf)
    pltpu.sync_copy(r0_hbm.at[pl.ds(base, TPW_B)], idx.at[0])
    pltpu.sync_copy(r1_hbm.at[pl.ds(base, TPW_B)], idx.at[1])
    d0 = pltpu.async_copy(xbuf, xs_hbm.at[idx.at[0]], sem)
    d1 = pltpu.async_copy(xbuf, xs_hbm.at[idx.at[1]], sem)
    d0.wait()
    d1.wait()


def _scatter_sc(x, r0, r1):
    return pl.kernel(
        _scatter_body,
        out_type=jax.ShapeDtypeStruct((MAXROWS, H), jnp.float32),
        mesh=plsc.VectorSubcoreMesh(**_MESH),
        scratch_types=[
            pltpu.VMEM((TPW_B, H), jnp.float32),
            pltpu.VMEM((2, TPW_B), jnp.int32),
            pltpu.SemaphoreType.DMA,
        ],
        compiler_params=_SC_PARAMS,
    )(x, r0, r1)


def _mlp_body(meta_ref, x_ref, w1_ref, w3_ref, w2_ref, out_ref, acc_ref):
    j = pl.program_id(0)
    i = pl.program_id(1)
    active = i < meta_ref[2 * L]

    @pl.when(active)
    def _():
        x = x_ref[...]
        g = lax.dot_general(x, w1_ref[0], (((1,), (1,)), ((), ())),
                            preferred_element_type=jnp.float32)
        u = lax.dot_general(x, w3_ref[0], (((1,), (1,)), ((), ())),
                            preferred_element_type=jnp.float32)
        h = g * jax.nn.sigmoid(g) * u
        contrib = lax.dot_general(h, w2_ref[0], (((1,), (1,)), ((), ())),
                                  preferred_element_type=jnp.float32)

        @pl.when(j == 0)
        def _():
            acc_ref[pl.ds(i * B, B), :] = contrib

        @pl.when(j > 0)
        def _():
            acc_ref[pl.ds(i * B, B), :] = acc_ref[pl.ds(i * B, B), :] + contrib

    @pl.when(jnp.logical_and(j == NC - 1, active))
    def _():
        out_ref[...] = acc_ref[pl.ds(i * B, B), :]


def _grouped_mlp(x_sorted, W1, W3, W2, meta):
    grid_spec = pltpu.PrefetchScalarGridSpec(
        num_scalar_prefetch=1,
        grid=(NC, NBLK),
        in_specs=[
            pl.BlockSpec((B, H),
                         lambda j, i, m: (jnp.minimum(i, m[2 * L] - 1), 0)),
            pl.BlockSpec((1, CH, H), lambda j, i, m: (m[i], j, 0)),
            pl.BlockSpec((1, CH, H), lambda j, i, m: (m[i], j, 0)),
            pl.BlockSpec((1, H, CH), lambda j, i, m: (m[i], 0, j)),
        ],
        out_specs=pl.BlockSpec((B, H),
                               lambda j, i, m: (jnp.where(j == NC - 1, i, 0),
                                                0)),
        scratch_shapes=[pltpu.VMEM((MAXROWS, H), jnp.float32)],
    )
    return pl.pallas_call(
        _mlp_body,
        grid_spec=grid_spec,
        out_shape=jax.ShapeDtypeStruct((MAXROWS, H), jnp.float32),
        compiler_params=pltpu.CompilerParams(
            dimension_semantics=("arbitrary", "arbitrary")),
    )(meta, x_sorted, W1, W3, W2)


_HB = TPW_B // 2


def _combine_body(rows_hbm, r0_hbm, r1_hbm, w0_hbm, w1_hbm, out_hbm,
                  idx0, idx1, wb0, wb1, buf0, buf1, sem):
    cid = lax.axis_index("c")
    sid = lax.axis_index("s")
    base = (sid * NCORES + cid) * TPW_B
    pltpu.sync_copy(r0_hbm.at[pl.ds(base, TPW_B)], idx0)
    pltpu.sync_copy(r1_hbm.at[pl.ds(base, TPW_B)], idx1)
    pltpu.sync_copy(w0_hbm.at[pl.ds(base, TPW_B)], wb0)
    pltpu.sync_copy(w1_hbm.at[pl.ds(base, TPW_B)], wb1)
    zero = jnp.zeros((L,), jnp.int32)
    for half in range(2):
        d0 = pltpu.async_copy(rows_hbm.at[idx0.at[pl.ds(half * _HB, _HB)]],
                              buf0, sem)
        d1 = pltpu.async_copy(rows_hbm.at[idx1.at[pl.ds(half * _HB, _HB)]],
                              buf1, sem)
        d0.wait()
        d1.wait()

        @pl.loop(0, _HB)
        def _(t):
            idxv = zero + (half * _HB + t)
            w0v = plsc.load_gather(wb0, [idxv])
            w1v = plsc.load_gather(wb1, [idxv])
            for l in range(H // L):
                sl = pl.ds(l * L, L)
                buf0[t, sl] = buf0[t, sl] * w0v + buf1[t, sl] * w1v

        pltpu.sync_copy(buf0, out_hbm.at[pl.ds(base + half * _HB, _HB)])


def _combine_sc(out_rows, r0, r1, w0, w1):
    return pl.kernel(
        _combine_body,
        out_type=jax.ShapeDtypeStruct((T, H), jnp.float32),
        mesh=plsc.VectorSubcoreMesh(**_MESH),
        scratch_types=[
            pltpu.VMEM((TPW_B,), jnp.int32),
            pltpu.VMEM((TPW_B,), jnp.int32),
            pltpu.VMEM((TPW_B,), jnp.float32),
            pltpu.VMEM((TPW_B,), jnp.float32),
            pltpu.VMEM((_HB, H), jnp.float32),
            pltpu.VMEM((_HB, H), jnp.float32),
            pltpu.SemaphoreType.DMA,
        ],
        compiler_params=_SC_PARAMS,
    )(out_rows, r0, r1, w0, w1)


def _dispatch_jnp(router_logits):
    topv, topi = jax.lax.top_k(router_logits, K)
    w = jax.nn.softmax(topv, axis=-1)
    flat_e = topi.reshape(-1).astype(jnp.int32)
    R = flat_e.shape[0]
    onehot = (flat_e[:, None] == jnp.arange(E, dtype=jnp.int32)[None, :]).astype(jnp.int32)
    ranks = jnp.cumsum(onehot, axis=0)
    myrank = jnp.take_along_axis(ranks, flat_e[:, None], axis=1)[:, 0] - 1
    counts = ranks[-1]
    padded = ((counts + B - 1) // B) * B
    poff = jnp.concatenate([jnp.zeros(1, jnp.int32),
                            jnp.cumsum(padded)[:-1].astype(jnp.int32)])
    dest = poff[flat_e] + myrank
    ends = ((poff + padded) // B).astype(jnp.int32)
    bid = jnp.arange(2 * L, dtype=jnp.int32)
    block_expert = jnp.minimum(
        jnp.sum((bid[:, None] >= ends[None, :]).astype(jnp.int32), axis=1),
        E - 1).astype(jnp.int32)
    nact = (jnp.sum(padded) // B).astype(jnp.int32)
    meta = jnp.concatenate([block_expert, jnp.full((L,), nact, jnp.int32)])
    tr = dest.reshape(T, K)
    return tr[:, 0], tr[:, 1], w[:, 0], w[:, 1], meta


def kernel(hidden_states, router_logits, W1, W3, W2):
    x = hidden_states.reshape(-1, H)
    logits_flat = router_logits.T.reshape(-1)
    r0, r1, w0, w1, meta = _route_sc(logits_flat)
    x_sorted = _scatter_sc(x, r0, r1)
    out_rows = _grouped_mlp(x_sorted, W1, W3, W2, meta)
    final = _combine_sc(out_rows, r0, r1, w0, w1)
    return final.reshape(hidden_states.shape)


# speedup vs baseline: 1.9995x; 1.0454x over previous
import functools
import jax
import jax.numpy as jnp
from jax import lax
from jax.experimental import pallas as pl
from jax.experimental.pallas import tpu as pltpu
from jax.experimental.pallas import tpu_sc as plsc

E = 8
K = 2
H = 1024
I = 4096
T = 2048
B = 256
NBLK = T * K // B + E
MAXROWS = NBLK * B
CH = 512
NC = I // CH

NCORES = 2
NSUB = 16
NW = NCORES * NSUB
L = 16

_MESH = dict(core_axis_name="c", subcore_axis_name="s",
             num_cores=NCORES, num_subcores=NSUB)
_SC_PARAMS = pltpu.CompilerParams(needs_layout_passes=False)


TPW_A = T // NW
NCH_A = TPW_A // L
NCHG = T // L


def _top2(lbuf, off):
    zero = jnp.zeros((L,), jnp.int32)
    v1 = lbuf[pl.ds(off, L)]
    i1 = zero
    v2 = jnp.full((L,), -jnp.inf, jnp.float32)
    i2 = zero
    for e in range(1, E):
        ve = lbuf[pl.ds(e * T + off, L)]
        gt1 = ve > v1
        gt2 = ve > v2
        i2 = jnp.where(gt1, i1, jnp.where(gt2, jnp.int32(e), i2))
        v2 = jnp.where(gt1, v1, jnp.where(gt2, ve, v2))
        i1 = jnp.where(gt1, jnp.int32(e), i1)
        v1 = jnp.where(gt1, ve, v1)
    return v1, i1, v2, i2


def _route_body(logits_flat, r0_hbm, r1_hbm, w0_hbm, w1_hbm, meta_hbm,
                lbuf, r0b, r1b, w0b, w1b, bebuf):
    cid = lax.axis_index("c")
    sid = lax.axis_index("s")
    wid = sid * NCORES + cid
    iota = lax.iota(jnp.int32, L)
    zero = jnp.zeros((L,), jnp.int32)
    pltpu.sync_copy(logits_flat, lbuf)

    my_first = wid * NCH_A

    @pl.loop(0, NCHG, init_carry=(zero, zero))
    def scan(c, carry):
        totals, myprefix = carry
        myprefix = jnp.where(c == my_first, totals, myprefix)
        _, i1, _, i2 = _top2(lbuf, c * L)
        for e in range(E):
            s = jnp.sum(jnp.where(jnp.logical_or(i1 == e, i2 == e),
                                  jnp.int32(1), jnp.int32(0)))
            totals = totals + jnp.where(iota == e, s, zero)
        return totals, myprefix

    totals, myprefix = scan
    padded = lax.shift_left(lax.shift_right_logical(totals + (B - 1), 8), 8)
    poff = plsc.cumsum(padded) - padded
    nact = lax.shift_right_logical(jnp.sum(padded), 8)

    cnt2 = poff + myprefix
    for c in range(NCH_A):
        off = (my_first + c) * L
        v1, i1, v2, i2 = _top2(lbuf, off)
        w0 = 1.0 / (1.0 + jnp.exp(v2 - v1))
        sl = pl.ds(c * L, L)
        w0b[sl] = w0
        w1b[sl] = 1.0 - w0
        r0 = zero
        r1 = zero
        for e in range(E):
            h1 = i1 == e
            h2 = i2 == e
            hi = jnp.where(jnp.logical_or(h1, h2),
                           jnp.int32(1), jnp.int32(0))
            pos = plsc.cumsum(hi) - hi
            d = pos + cnt2[e]
            r0 = jnp.where(h1, d, r0)
            r1 = jnp.where(h2, d, r1)
            cnt2 = cnt2 + jnp.where(iota == e, jnp.sum(hi), zero)
        r0b[sl] = r0
        r1b[sl] = r1

    base_t = wid * TPW_A
    pltpu.sync_copy(r0b, r0_hbm.at[pl.ds(base_t, TPW_A)])
    pltpu.sync_copy(r1b, r1_hbm.at[pl.ds(base_t, TPW_A)])
    pltpu.sync_copy(w0b, w0_hbm.at[pl.ds(base_t, TPW_A)])
    pltpu.sync_copy(w1b, w1_hbm.at[pl.ds(base_t, TPW_A)])

    @pl.when(wid == 0)
    def _():
        ends = lax.shift_right_logical(poff + padded, 8)
        for bc in range(2):
            lanes = iota + bc * L
            bev = zero
            for e in range(E):
                bev = bev + jnp.where(lanes >= ends[e],
                                      jnp.int32(1), jnp.int32(0))
            bebuf[pl.ds(bc * L, L)] = jnp.minimum(bev, E - 1)
        bebuf[pl.ds(2 * L, L)] = jnp.where(iota == 0, nact, zero)
        pltpu.sync_copy(bebuf, meta_hbm)


def _route_sc(logits_flat):
    return pl.kernel(
        _route_body,
        out_type=(
            jax.ShapeDtypeStruct((T,), jnp.int32),
            jax.ShapeDtypeStruct((T,), jnp.int32),
            jax.ShapeDtypeStruct((T,), jnp.float32),
            jax.ShapeDtypeStruct((T,), jnp.float32),
            jax.ShapeDtypeStruct((3 * L,), jnp.int32),
        ),
        mesh=plsc.VectorSubcoreMesh(**_MESH),
        scratch_types=[
            pltpu.VMEM((E * T,), jnp.float32),
            pltpu.VMEM((TPW_A,), jnp.int32),
            pltpu.VMEM((TPW_A,), jnp.int32),
            pltpu.VMEM((TPW_A,), jnp.float32),
            pltpu.VMEM((TPW_A,), jnp.float32),
            pltpu.VMEM((3 * L,), jnp.int32),
        ],
        compiler_params=_SC_PARAMS,
    )(logits_flat)


TPW_B = T // NW


def _scatter_body(hidden, r0_hbm, r1_hbm, xs_hbm, xbuf, idx, sem):
    cid = lax.axis_index("c")
    sid = lax.axis_index("s")
    base = (sid * NCORES + cid) * TPW_B
    pltpu.sync_copy(hidden.at[pl.ds(base, TPW_B)], xbuf)
    pltpu.sync_copy(r0_hbm.at[pl.ds(base, TPW_B)], idx.at[0])
    pltpu.sync_copy(r1_hbm.at[pl.ds(base, TPW_B)], idx.at[1])
    d0 = pltpu.async_copy(xbuf, xs_hbm.at[idx.at[0]], sem)
    d1 = pltpu.async_copy(xbuf, xs_hbm.at[idx.at[1]], sem)
    d0.wait()
    d1.wait()


def _scatter_sc(x, r0, r1):
    return pl.kernel(
        _scatter_body,
        out_type=jax.ShapeDtypeStruct((MAXROWS, H), jnp.float32),
        mesh=plsc.VectorSubcoreMesh(**_MESH),
        scratch_types=[
            pltpu.VMEM((TPW_B, H), jnp.float32),
            pltpu.VMEM((2, TPW_B), jnp.int32),
            pltpu.SemaphoreType.DMA,
        ],
        compiler_params=_SC_PARAMS,
    )(x, r0, r1)


_WBUF = pl.Buffered(buffer_count=4, use_lookahead=True)


def _mlp_outer(meta_ref, x_hbm, w1_hbm, w3_hbm, w2_hbm, out_hbm, acc_ref):
    def body(idx, x_ref, w1_ref, w3_ref, w2_ref, out_ref):
        j, i = idx
        active = i < meta_ref[2 * L]

        @pl.when(active)
        def _():
            x = x_ref[...]
            g = lax.dot_general(x, w1_ref[0], (((1,), (1,)), ((), ())),
                                preferred_element_type=jnp.float32)
            u = lax.dot_general(x, w3_ref[0], (((1,), (1,)), ((), ())),
                                preferred_element_type=jnp.float32)
            h = g * jax.nn.sigmoid(g) * u
            contrib = lax.dot_general(h, w2_ref[0], (((1,), (1,)), ((), ())),
                                      preferred_element_type=jnp.float32)

            @pl.when(j == 0)
            def _():
                acc_ref[pl.ds(i * B, B), :] = contrib

            @pl.when(j > 0)
            def _():
                acc_ref[pl.ds(i * B, B), :] = (acc_ref[pl.ds(i * B, B), :]
                                               + contrib)

        @pl.when(jnp.logical_and(j == NC - 1, active))
        def _():
            out_ref[...] = acc_ref[pl.ds(i * B, B), :]

    pipe = pltpu.emit_pipeline(
        body,
        grid=(NC, NBLK),
        in_specs=[
            pl.BlockSpec((B, H),
                         lambda j, i: (jnp.minimum(i, meta_ref[2 * L] - 1),
                                       0)),
            pl.BlockSpec((1, CH, H), lambda j, i: (meta_ref[i], j, 0),
                         pipeline_mode=_WBUF),
            pl.BlockSpec((1, CH, H), lambda j, i: (meta_ref[i], j, 0),
                         pipeline_mode=_WBUF),
            pl.BlockSpec((1, H, CH), lambda j, i: (meta_ref[i], 0, j),
                         pipeline_mode=_WBUF),
        ],
        out_specs=[
            pl.BlockSpec((B, H),
                         lambda j, i: (jnp.where(j == NC - 1, i, 0), 0)),
        ],
        _explicit_indices=True,
    )
    pipe(x_hbm, w1_hbm, w3_hbm, w2_hbm, out_hbm)


def _grouped_mlp(x_sorted, W1, W3, W2, meta):
    return pl.pallas_call(
        _mlp_outer,
        in_specs=[
            pl.BlockSpec(memory_space=pltpu.SMEM),
            pl.BlockSpec(memory_space=pl.ANY),
            pl.BlockSpec(memory_space=pl.ANY),
            pl.BlockSpec(memory_space=pl.ANY),
            pl.BlockSpec(memory_space=pl.ANY),
        ],
        out_specs=pl.BlockSpec(memory_space=pl.ANY),
        scratch_shapes=[pltpu.VMEM((MAXROWS, H), jnp.float32)],
        out_shape=jax.ShapeDtypeStruct((MAXROWS, H), jnp.float32),
        compiler_params=pltpu.CompilerParams(
            vmem_limit_bytes=100 * 1024 * 1024),
    )(meta, x_sorted, W1, W3, W2)


_HB = TPW_B // 2


def _combine_body(rows_hbm, r0_hbm, r1_hbm, w0_hbm, w1_hbm, out_hbm,
                  idx0, idx1, wb0, wb1, buf0, buf1, sem):
    cid = lax.axis_index("c")
    sid = lax.axis_index("s")
    base = (sid * NCORES + cid) * TPW_B
    pltpu.sync_copy(r0_hbm.at[pl.ds(base, TPW_B)], idx0)
    pltpu.sync_copy(r1_hbm.at[pl.ds(base, TPW_B)], idx1)
    pltpu.sync_copy(w0_hbm.at[pl.ds(base, TPW_B)], wb0)
    pltpu.sync_copy(w1_hbm.at[pl.ds(base, TPW_B)], wb1)
    zero = jnp.zeros((L,), jnp.int32)
    for half in range(2):
        d0 = pltpu.async_copy(rows_hbm.at[idx0.at[pl.ds(half * _HB, _HB)]],
                              buf0, sem)
        d1 = pltpu.async_copy(rows_hbm.at[idx1.at[pl.ds(half * _HB, _HB)]],
                              buf1, sem)
        d0.wait()
        d1.wait()

        @pl.loop(0, _HB)
        def _(t):
            idxv = zero + (half * _HB + t)
            w0v = plsc.load_gather(wb0, [idxv])
            w1v = plsc.load_gather(wb1, [idxv])
            for l in range(H // L):
                sl = pl.ds(l * L, L)
                buf0[t, sl] = buf0[t, sl] * w0v + buf1[t, sl] * w1v

        pltpu.sync_copy(buf0, out_hbm.at[pl.ds(base + half * _HB, _HB)])


def _combine_sc(out_rows, r0, r1, w0, w1):
    return pl.kernel(
        _combine_body,
        out_type=jax.ShapeDtypeStruct((T, H), jnp.float32),
        mesh=plsc.VectorSubcoreMesh(**_MESH),
        scratch_types=[
            pltpu.VMEM((TPW_B,), jnp.int32),
            pltpu.VMEM((TPW_B,), jnp.int32),
            pltpu.VMEM((TPW_B,), jnp.float32),
            pltpu.VMEM((TPW_B,), jnp.float32),
            pltpu.VMEM((_HB, H), jnp.float32),
            pltpu.VMEM((_HB, H), jnp.float32),
            pltpu.SemaphoreType.DMA,
        ],
        compiler_params=_SC_PARAMS,
    )(out_rows, r0, r1, w0, w1)


def _dispatch_jnp(router_logits):
    topv, topi = jax.lax.top_k(router_logits, K)
    w = jax.nn.softmax(topv, axis=-1)
    flat_e = topi.reshape(-1).astype(jnp.int32)
    R = flat_e.shape[0]
    onehot = (flat_e[:, None] == jnp.arange(E, dtype=jnp.int32)[None, :]).astype(jnp.int32)
    ranks = jnp.cumsum(onehot, axis=0)
    myrank = jnp.take_along_axis(ranks, flat_e[:, None], axis=1)[:, 0] - 1
    counts = ranks[-1]
    padded = ((counts + B - 1) // B) * B
    poff = jnp.concatenate([jnp.zeros(1, jnp.int32),
                            jnp.cumsum(padded)[:-1].astype(jnp.int32)])
    dest = poff[flat_e] + myrank
    ends = ((poff + padded) // B).astype(jnp.int32)
    bid = jnp.arange(2 * L, dtype=jnp.int32)
    block_expert = jnp.minimum(
        jnp.sum((bid[:, None] >= ends[None, :]).astype(jnp.int32), axis=1),
        E - 1).astype(jnp.int32)
    nact = (jnp.sum(padded) // B).astype(jnp.int32)
    meta = jnp.concatenate([block_expert, jnp.full((L,), nact, jnp.int32)])
    tr = dest.reshape(T, K)
    return tr[:, 0], tr[:, 1], w[:, 0], w[:, 1], meta


def kernel(hidden_states, router_logits, W1, W3, W2):
    x = hidden_states.reshape(-1, H)
    logits_flat = router_logits.T.reshape(-1)
    r0, r1, w0, w1, meta = _route_sc(logits_flat)
    x_sorted = _scatter_sc(x, r0, r1)
    out_rows = _grouped_mlp(x_sorted, W1, W3, W2, meta)
    final = _combine_sc(out_rows, r0, r1, w0, w1)
    return final.reshape(hidden_states.shape)


# speedup vs baseline: 2.1013x; 1.0509x over previous
import functools
import jax
import jax.numpy as jnp
from jax import lax
from jax.experimental import pallas as pl
from jax.experimental.pallas import tpu as pltpu
from jax.experimental.pallas import tpu_sc as plsc

E = 8
K = 2
H = 1024
I = 4096
T = 2048
B = 256
NBLK = T * K // B + E
MAXROWS = NBLK * B
CH = 512
NC = I // CH

NCORES = 2
NSUB = 16
NW = NCORES * NSUB
L = 16

_MESH = dict(core_axis_name="c", subcore_axis_name="s",
             num_cores=NCORES, num_subcores=NSUB)
_SC_PARAMS = pltpu.CompilerParams(needs_layout_passes=False)


TPW_A = T // NW
NCH_A = TPW_A // L
NCHG = T // L


def _top2(lbuf, off):
    zero = jnp.zeros((L,), jnp.int32)
    v1 = lbuf[pl.ds(off, L)]
    i1 = zero
    v2 = jnp.full((L,), -jnp.inf, jnp.float32)
    i2 = zero
    for e in range(1, E):
        ve = lbuf[pl.ds(e * T + off, L)]
        gt1 = ve > v1
        gt2 = ve > v2
        i2 = jnp.where(gt1, i1, jnp.where(gt2, jnp.int32(e), i2))
        v2 = jnp.where(gt1, v1, jnp.where(gt2, ve, v2))
        i1 = jnp.where(gt1, jnp.int32(e), i1)
        v1 = jnp.where(gt1, ve, v1)
    return v1, i1, v2, i2


def _route_body(logits_flat, r0_hbm, r1_hbm, w0_hbm, w1_hbm, meta_hbm,
                lbuf, r0b, r1b, w0b, w1b, bebuf):
    cid = lax.axis_index("c")
    sid = lax.axis_index("s")
    wid = sid * NCORES + cid
    iota = lax.iota(jnp.int32, L)
    zero = jnp.zeros((L,), jnp.int32)
    pltpu.sync_copy(logits_flat, lbuf)

    my_first = wid * NCH_A

    @pl.loop(0, NCHG, init_carry=(zero, zero))
    def scan(c, carry):
        totals, myprefix = carry
        myprefix = jnp.where(c == my_first, totals, myprefix)
        _, i1, _, i2 = _top2(lbuf, c * L)
        for e in range(E):
            s = jnp.sum(jnp.where(jnp.logical_or(i1 == e, i2 == e),
                                  jnp.int32(1), jnp.int32(0)))
            totals = totals + jnp.where(iota == e, s, zero)
        return totals, myprefix

    totals, myprefix = scan
    padded = lax.shift_left(lax.shift_right_logical(totals + (B - 1), 8), 8)
    poff = plsc.cumsum(padded) - padded
    nact = lax.shift_right_logical(jnp.sum(padded), 8)

    cnt2 = poff + myprefix
    for c in range(NCH_A):
        off = (my_first + c) * L
        v1, i1, v2, i2 = _top2(lbuf, off)
        w0 = 1.0 / (1.0 + jnp.exp(v2 - v1))
        sl = pl.ds(c * L, L)
        w0b[sl] = w0
        w1b[sl] = 1.0 - w0
        r0 = zero
        r1 = zero
        for e in range(E):
            h1 = i1 == e
            h2 = i2 == e
            hi = jnp.where(jnp.logical_or(h1, h2),
                           jnp.int32(1), jnp.int32(0))
            pos = plsc.cumsum(hi) - hi
            d = pos + cnt2[e]
            r0 = jnp.where(h1, d, r0)
            r1 = jnp.where(h2, d, r1)
            cnt2 = cnt2 + jnp.where(iota == e, jnp.sum(hi), zero)
        r0b[sl] = r0
        r1b[sl] = r1

    base_t = wid * TPW_A
    pltpu.sync_copy(r0b, r0_hbm.at[pl.ds(base_t, TPW_A)])
    pltpu.sync_copy(r1b, r1_hbm.at[pl.ds(base_t, TPW_A)])
    pltpu.sync_copy(w0b, w0_hbm.at[pl.ds(base_t, TPW_A)])
    pltpu.sync_copy(w1b, w1_hbm.at[pl.ds(base_t, TPW_A)])

    @pl.when(wid == 0)
    def _():
        ends = lax.shift_right_logical(poff + padded, 8)
        for bc in range(2):
            lanes = iota + bc * L
            bev = zero
            for e in range(E):
                bev = bev + jnp.where(lanes >= ends[e],
                                      jnp.int32(1), jnp.int32(0))
            bebuf[pl.ds(bc * L, L)] = jnp.minimum(bev, E - 1)
        bebuf[pl.ds(2 * L, L)] = jnp.where(iota == 0, nact, zero)
        pltpu.sync_copy(bebuf, meta_hbm)


def _route_sc(logits_flat):
    return pl.kernel(
        _route_body,
        out_type=(
            jax.ShapeDtypeStruct((T,), jnp.int32),
            jax.ShapeDtypeStruct((T,), jnp.int32),
            jax.ShapeDtypeStruct((T,), jnp.float32),
            jax.ShapeDtypeStruct((T,), jnp.float32),
            jax.ShapeDtypeStruct((3 * L,), jnp.int32),
        ),
        mesh=plsc.VectorSubcoreMesh(**_MESH),
        scratch_types=[
            pltpu.VMEM((E * T,), jnp.float32),
            pltpu.VMEM((TPW_A,), jnp.int32),
            pltpu.VMEM((TPW_A,), jnp.int32),
            pltpu.VMEM((TPW_A,), jnp.float32),
            pltpu.VMEM((TPW_A,), jnp.float32),
            pltpu.VMEM((3 * L,), jnp.int32),
        ],
        compiler_params=_SC_PARAMS,
    )(logits_flat)


TPW_B = T // NW


def _scatter_body(hidden, r0_hbm, r1_hbm, xs_hbm, xbuf, idx, sem):
    cid = lax.axis_index("c")
    sid = lax.axis_index("s")
    base = (sid * NCORES + cid) * TPW_B
    pltpu.sync_copy(hidden.at[pl.ds(base, TPW_B)], xbuf)
    pltpu.sync_copy(r0_hbm.at[pl.ds(base, TPW_B)], idx.at[0])
    pltpu.sync_copy(r1_hbm.at[pl.ds(base, TPW_B)], idx.at[1])
    d0 = pltpu.async_copy(xbuf, xs_hbm.at[idx.at[0]], sem)
    d1 = pltpu.async_copy(xbuf, xs_hbm.at[idx.at[1]], sem)
    d0.wait()
    d1.wait()


def _scatter_sc(x, r0, r1):
    return pl.kernel(
        _scatter_body,
        out_type=jax.ShapeDtypeStruct((MAXROWS, H), jnp.float32),
        mesh=plsc.VectorSubcoreMesh(**_MESH),
        scratch_types=[
            pltpu.VMEM((TPW_B, H), jnp.float32),
            pltpu.VMEM((2, TPW_B), jnp.int32),
            pltpu.SemaphoreType.DMA,
        ],
        compiler_params=_SC_PARAMS,
    )(x, r0, r1)


_WBUF = pl.Buffered(buffer_count=5, use_lookahead=True)
_XBUF = pl.Buffered(buffer_count=3, use_lookahead=True)


def _mlp_outer(meta_ref, x_hbm, w1_hbm, w3_hbm, w2_hbm, out_hbm, acc_ref):
    def body(idx, x_ref, w1_ref, w3_ref, w2_ref, out_ref):
        j, i = idx
        active = i < meta_ref[2 * L]

        @pl.when(active)
        def _():
            x = x_ref[...]
            g = lax.dot_general(x, w1_ref[0], (((1,), (1,)), ((), ())),
                                preferred_element_type=jnp.float32)
            u = lax.dot_general(x, w3_ref[0], (((1,), (1,)), ((), ())),
                                preferred_element_type=jnp.float32)
            h = g * jax.nn.sigmoid(g) * u
            contrib = lax.dot_general(h, w2_ref[0], (((1,), (1,)), ((), ())),
                                      preferred_element_type=jnp.float32)

            @pl.when(j == 0)
            def _():
                acc_ref[pl.ds(i * B, B), :] = contrib

            @pl.when(j > 0)
            def _():
                acc_ref[pl.ds(i * B, B), :] = (acc_ref[pl.ds(i * B, B), :]
                                               + contrib)

        @pl.when(jnp.logical_and(j == NC - 1, active))
        def _():
            out_ref[...] = acc_ref[pl.ds(i * B, B), :]

    pipe = pltpu.emit_pipeline(
        body,
        grid=(NC, NBLK),
        in_specs=[
            pl.BlockSpec((B, H),
                         lambda j, i: (jnp.minimum(i, meta_ref[2 * L] - 1),
                                       0),
                         pipeline_mode=_XBUF),
            pl.BlockSpec((1, CH, H), lambda j, i: (meta_ref[i], j, 0),
                         pipeline_mode=_WBUF),
            pl.BlockSpec((1, CH, H), lambda j, i: (meta_ref[i], j, 0),
                         pipeline_mode=_WBUF),
            pl.BlockSpec((1, H, CH), lambda j, i: (meta_ref[i], 0, j),
                         pipeline_mode=_WBUF),
        ],
        out_specs=[
            pl.BlockSpec((B, H),
                         lambda j, i: (jnp.where(j == NC - 1, i, 0), 0)),
        ],
        _explicit_indices=True,
    )
    pipe(x_hbm, w1_hbm, w3_hbm, w2_hbm, out_hbm)


def _grouped_mlp(x_sorted, W1, W3, W2, meta):
    return pl.pallas_call(
        _mlp_outer,
        in_specs=[
            pl.BlockSpec(memory_space=pltpu.SMEM),
            pl.BlockSpec(memory_space=pl.ANY),
            pl.BlockSpec(memory_space=pl.ANY),
            pl.BlockSpec(memory_space=pl.ANY),
            pl.BlockSpec(memory_space=pl.ANY),
        ],
        out_specs=pl.BlockSpec(memory_space=pl.ANY),
        scratch_shapes=[pltpu.VMEM((MAXROWS, H), jnp.float32)],
        out_shape=jax.ShapeDtypeStruct((MAXROWS, H), jnp.float32),
        compiler_params=pltpu.CompilerParams(
            vmem_limit_bytes=100 * 1024 * 1024),
    )(meta, x_sorted, W1, W3, W2)


_HB = TPW_B // 2


def _combine_body(rows_hbm, r0_hbm, r1_hbm, w0_hbm, w1_hbm, out_hbm,
                  idx0, idx1, wb0, wb1, buf0, buf1, sem):
    cid = lax.axis_index("c")
    sid = lax.axis_index("s")
    base = (sid * NCORES + cid) * TPW_B
    pltpu.sync_copy(r0_hbm.at[pl.ds(base, TPW_B)], idx0)
    pltpu.sync_copy(r1_hbm.at[pl.ds(base, TPW_B)], idx1)
    pltpu.sync_copy(w0_hbm.at[pl.ds(base, TPW_B)], wb0)
    pltpu.sync_copy(w1_hbm.at[pl.ds(base, TPW_B)], wb1)
    zero = jnp.zeros((L,), jnp.int32)
    for half in range(2):
        d0 = pltpu.async_copy(rows_hbm.at[idx0.at[pl.ds(half * _HB, _HB)]],
                              buf0, sem)
        d1 = pltpu.async_copy(rows_hbm.at[idx1.at[pl.ds(half * _HB, _HB)]],
                              buf1, sem)
        d0.wait()
        d1.wait()

        @pl.loop(0, _HB)
        def _(t):
            idxv = zero + (half * _HB + t)
            w0v = plsc.load_gather(wb0, [idxv])
            w1v = plsc.load_gather(wb1, [idxv])
            for l in range(H // L):
                sl = pl.ds(l * L, L)
                buf0[t, sl] = buf0[t, sl] * w0v + buf1[t, sl] * w1v

        pltpu.sync_copy(buf0, out_hbm.at[pl.ds(base + half * _HB, _HB)])


def _combine_sc(out_rows, r0, r1, w0, w1):
    return pl.kernel(
        _combine_body,
        out_type=jax.ShapeDtypeStruct((T, H), jnp.float32),
        mesh=plsc.VectorSubcoreMesh(**_MESH),
        scratch_types=[
            pltpu.VMEM((TPW_B,), jnp.int32),
            pltpu.VMEM((TPW_B,), jnp.int32),
            pltpu.VMEM((TPW_B,), jnp.float32),
            pltpu.VMEM((TPW_B,), jnp.float32),
            pltpu.VMEM((_HB, H), jnp.float32),
            pltpu.VMEM((_HB, H), jnp.float32),
            pltpu.SemaphoreType.DMA,
        ],
        compiler_params=_SC_PARAMS,
    )(out_rows, r0, r1, w0, w1)


def _dispatch_jnp(router_logits):
    topv, topi = jax.lax.top_k(router_logits, K)
    w = jax.nn.softmax(topv, axis=-1)
    flat_e = topi.reshape(-1).astype(jnp.int32)
    R = flat_e.shape[0]
    onehot = (flat_e[:, None] == jnp.arange(E, dtype=jnp.int32)[None, :]).astype(jnp.int32)
    ranks = jnp.cumsum(onehot, axis=0)
    myrank = jnp.take_along_axis(ranks, flat_e[:, None], axis=1)[:, 0] - 1
    counts = ranks[-1]
    padded = ((counts + B - 1) // B) * B
    poff = jnp.concatenate([jnp.zeros(1, jnp.int32),
                            jnp.cumsum(padded)[:-1].astype(jnp.int32)])
    dest = poff[flat_e] + myrank
    ends = ((poff + padded) // B).astype(jnp.int32)
    bid = jnp.arange(2 * L, dtype=jnp.int32)
    block_expert = jnp.minimum(
        jnp.sum((bid[:, None] >= ends[None, :]).astype(jnp.int32), axis=1),
        E - 1).astype(jnp.int32)
    nact = (jnp.sum(padded) // B).astype(jnp.int32)
    meta = jnp.concatenate([block_expert, jnp.full((L,), nact, jnp.int32)])
    tr = dest.reshape(T, K)
    return tr[:, 0], tr[:, 1], w[:, 0], w[:, 1], meta


def kernel(hidden_states, router_logits, W1, W3, W2):
    x = hidden_states.reshape(-1, H)
    logits_flat = router_logits.T.reshape(-1)
    r0, r1, w0, w1, meta = _route_sc(logits_flat)
    x_sorted = _scatter_sc(x, r0, r1)
    out_rows = _grouped_mlp(x_sorted, W1, W3, W2, meta)
    final = _combine_sc(out_rows, r0, r1, w0, w1)
    return final.reshape(hidden_states.shape)
